# Initial kernel scaffold; baseline (speedup 1.0000x reference)
import jax
import jax.numpy as jnp
from jax.experimental import pallas as pl


def kernel(params, x, edge_index, edge_attr):
    raise NotImplementedError("write your pallas kernel here")


# speedup vs baseline: 1.7520x; 1.7520x over previous
import functools

import jax
import jax.numpy as jnp
from jax import lax
from jax.experimental import pallas as pl
from jax.experimental.pallas import tpu as pltpu
from jax.experimental.pallas import tpu_sc as plsc

N = 10000
E = 320000
D = 128
L = 5
_ATOM_DIMS = [119, 4, 12, 12, 10, 6, 6, 2, 2]
_BOND_DIMS = [5, 6, 2]
K_ATOM = 176

NC, NS = 2, 16
NW = NC * NS
EPW = E // NW
C = 80
ROUNDS = EPW // C
NP = 10240
RPT = NP // NS
ZR = 32


def _sc_msgpass(h, srcs, combs, dsts, t60):
    mesh = plsc.VectorSubcoreMesh(core_axis_name="c", subcore_axis_name="s")

    @functools.partial(
        pl.kernel,
        out_type=jax.ShapeDtypeStruct((NC, NP, D), jnp.float32),
        mesh=mesh,
        scratch_types=[
            pltpu.VMEM((C,), jnp.int32),
            pltpu.VMEM((C,), jnp.int32),
            pltpu.VMEM((C,), jnp.int32),
            pltpu.VMEM((C, D), jnp.float32),
            pltpu.VMEM((C, D), jnp.float32),
            pltpu.VMEM((ZR, D), jnp.float32),
            pltpu.VMEM_SHARED((NP, D), jnp.float32),
            pltpu.SemaphoreType.DMA,
            pltpu.SemaphoreType.DMA,
        ],
    )
    def run(h_ref, src_ref, comb_ref, dst_ref, t_ref, out_ref,
            srcv, combv, dstv, av, bv, zv, aggs, sem_a, sem_b):
        cid = lax.axis_index("c")
        sid = lax.axis_index("s")
        wid = sid * NC + cid

        zero16 = jnp.zeros((16,), jnp.float32)

        def zrow(i, carry):
            for j in range(D // 16):
                zv[i, pl.ds(j * 16, 16)] = zero16
            return carry

        lax.fori_loop(0, ZR, zrow, 0)

        def zstripe(k, carry):
            pltpu.sync_copy(zv, aggs.at[pl.ds(sid * RPT + k * ZR, ZR)])
            return carry

        lax.fori_loop(0, RPT // ZR, zstripe, 0)
        plsc.subcore_barrier()

        def round_(r, carry):
            base = wid * EPW + r * C
            pltpu.sync_copy(src_ref.at[pl.ds(base, C)], srcv)
            pltpu.sync_copy(comb_ref.at[pl.ds(base, C)], combv)
            pltpu.sync_copy(dst_ref.at[pl.ds(base, C)], dstv)
            ga = pltpu.async_copy(h_ref.at[srcv], av, sem_a)
            gb = pltpu.async_copy(t_ref.at[combv], bv, sem_b)
            ga.wait()
            gb.wait()

            def crow(i, inner):
                for j in range(D // 16):
                    u = av[i, pl.ds(j * 16, 16)]
                    v = bv[i, pl.ds(j * 16, 16)]
                    av[i, pl.ds(j * 16, 16)] = jnp.maximum(u + v, 0.0)
                return inner

            lax.fori_loop(0, C, crow, 0)
            pltpu.sync_copy(av, aggs.at[dstv], add=True)
            return carry

        lax.fori_loop(0, ROUNDS, round_, 0)
        plsc.subcore_barrier()
        pltpu.sync_copy(aggs.at[pl.ds(sid * RPT, RPT)],
                        out_ref.at[cid, pl.ds(sid * RPT, RPT)])

    return run(h, srcs, combs, dsts, t60)


def _atom_body(x_ref, t_ref, o_ref):
    xv = x_ref[...]
    iota = lax.broadcasted_iota(jnp.int32, (N, K_ATOM), 1)
    acc = jnp.zeros((N, K_ATOM), jnp.float32)
    off = 0
    for i, d in enumerate(_ATOM_DIMS):
        col = xv[:, i][:, None] + off
        acc = acc + (iota == col).astype(jnp.float32)
        off += d
    o_ref[...] = jnp.dot(acc, t_ref[...], precision=lax.Precision.HIGHEST,
                         preferred_element_type=jnp.float32)


def _atom_encode(x, tabs):
    return pl.pallas_call(
        _atom_body,
        out_shape=jax.ShapeDtypeStruct((N, D), jnp.float32),
    )(x, tabs)


def _mlp_body(h_ref, p_ref, eps_ref, w1_ref, b1_ref, g1_ref, bt1_ref,
              w2_ref, b2_ref, g_ref, bt_ref, o_ref, *, relu_out):
    h = h_ref[...]
    agg = p_ref[0, :N] + p_ref[1, :N]
    z = (1.0 + eps_ref[0, 0]) * h + agg
    z1 = jnp.dot(z, w1_ref[...], preferred_element_type=jnp.float32) + b1_ref[...]
    mu = jnp.mean(z1, axis=0, keepdims=True)
    xc = z1 - mu
    var = jnp.mean(xc * xc, axis=0, keepdims=True)
    z1n = g1_ref[...] * xc / jnp.sqrt(var + 1e-5) + bt1_ref[...]
    a = jnp.maximum(z1n, 0.0)
    z2 = jnp.dot(a, w2_ref[...], preferred_element_type=jnp.float32) + b2_ref[...]
    mu2 = jnp.mean(z2, axis=0, keepdims=True)
    xc2 = z2 - mu2
    var2 = jnp.mean(xc2 * xc2, axis=0, keepdims=True)
    out = g_ref[...] * xc2 / jnp.sqrt(var2 + 1e-5) + bt_ref[...]
    if relu_out:
        out = jnp.maximum(out, 0.0)
    o_ref[...] = out


def _mlp(h, parts, p, relu_out):
    body = functools.partial(_mlp_body, relu_out=relu_out)
    return pl.pallas_call(
        body,
        out_shape=jax.ShapeDtypeStruct((N, D), jnp.float32),
    )(h, parts, p['eps'].reshape(1, 1), p['W1'], p['b1'].reshape(1, 2 * D),
      p['g1'].reshape(1, 2 * D), p['bt1'].reshape(1, 2 * D), p['W2'],
      p['b2'].reshape(1, D), p['g'].reshape(1, D), p['bt'].reshape(1, D))


def kernel(params, x, edge_index, edge_attr):
    atom_tab = jnp.concatenate(params['atom'], axis=0)
    atom_tab = jnp.pad(atom_tab, ((0, K_ATOM - atom_tab.shape[0]), (0, 0)))
    h = _atom_encode(x, atom_tab)

    src = edge_index[0]
    dst = edge_index[1]
    comb = (edge_attr[:, 0] * (_BOND_DIMS[1] * _BOND_DIMS[2])
            + edge_attr[:, 1] * _BOND_DIMS[2]
            + edge_attr[:, 2]).astype(jnp.int32)

    for l in range(L):
        p = params['layers'][l]
        b0, b1, b2 = p['bond']
        t60 = (b0[:, None, None, :] + b1[None, :, None, :]
               + b2[None, None, :, :]).reshape(-1, D)
        parts = _sc_msgpass(h, src, comb, dst, t60)
        h = _mlp(h, parts, p, relu_out=(l < L - 1))
    return h
